# baseline (speedup 1.0000x reference)
import functools

import jax
import jax.numpy as jnp
from jax import lax
from jax.experimental import pallas as pl
from jax.experimental.pallas import tpu as pltpu
from jax.experimental.pallas import tpu_sc as plsc

_H = 112
_W = 112
_NW = 32


def _sc_body(x_hbm, out_hbm, inbuf, obuf):
    wid = lax.axis_index("s") * 2 + lax.axis_index("c")
    lane = lax.broadcasted_iota(jnp.int32, (16,), 0)
    zeros16 = jnp.zeros((16,), jnp.float32)

    def zero_body(i, carry):
        def zcol(j, c2):
            obuf[i, pl.ds(j * 16, 16)] = zeros16
            return c2
        return lax.fori_loop(0, 4 * _W // 16, zcol, carry)

    lax.fori_loop(0, _H, zero_body, 0)

    n_imgs = 1536 // _NW

    def img_body(t, carry):
        img = wid * n_imgs + t
        pltpu.sync_copy(x_hbm.at[img], inbuf)

        def row_body(h, c):
            hvec = jnp.full((16,), h, jnp.int32)
            def vec_body(j, c2):
                v = inbuf[h, pl.ds(j * 16, 16)]
                idx = (2 * _W + 1 + 32 * j) + 2 * lane
                plsc.store_scatter(obuf, [hvec, idx], v)
                return c2
            return lax.fori_loop(0, _W // 16, vec_body, c)

        lax.fori_loop(0, _H, row_body, 0)
        pltpu.sync_copy(obuf, out_hbm.at[img])
        return carry

    lax.fori_loop(0, n_imgs, img_body, 0)


def kernel(x):
    B, I, C, H, W = x.shape
    n = B * I * C
    xf = x.reshape(n, H, W)
    mesh = plsc.VectorSubcoreMesh(core_axis_name="c", subcore_axis_name="s")
    out = pl.kernel(
        _sc_body,
        out_type=jax.ShapeDtypeStruct((n, H, 4 * W), jnp.float32),
        mesh=mesh,
        scratch_types=[
            pltpu.VMEM((H, W), jnp.float32),
            pltpu.VMEM((H, 4 * W), jnp.float32),
        ],
        compiler_params=pltpu.CompilerParams(needs_layout_passes=False),
    )(xf)
    return out.reshape(B, I, C, 2 * H, 2 * W)


# speedup vs baseline: 1.1730x; 1.1730x over previous
import functools

import jax
import jax.numpy as jnp
from jax import lax
from jax.experimental import pallas as pl
from jax.experimental.pallas import tpu as pltpu
from jax.experimental.pallas import tpu_sc as plsc

_H = 112
_W = 112
_IMG_IN = _H * _W
_IMG_OUT = 4 * _IMG_IN
_NW = 32
_N_IMG = 1536 // _NW


def _sc_body(x_hbm, out_hbm, inbuf0, inbuf1, obuf0, obuf1,
             sem_in0, sem_in1, sem_out0, sem_out1):
    inbufs = (inbuf0, inbuf1)
    obufs = (obuf0, obuf1)
    sem_ins = (sem_in0, sem_in1)
    sem_outs = (sem_out0, sem_out1)

    wid = lax.axis_index("s") * 2 + lax.axis_index("c")
    base_img = wid * _N_IMG
    lane2 = 2 * lax.broadcasted_iota(jnp.int32, (16,), 0)
    zeros16 = jnp.zeros((16,), jnp.float32)

    def zero_body(i, carry):
        obuf0[pl.ds(i * 16, 16)] = zeros16
        obuf1[pl.ds(i * 16, 16)] = zeros16
        return carry

    lax.fori_loop(0, _IMG_OUT // 16, zero_body, 0)

    pltpu.async_copy(x_hbm.at[base_img], inbuf0, sem_in0)
    pltpu.async_copy(x_hbm.at[base_img + 1], inbuf1, sem_in1)

    def scatter_image(ibuf, obuf):
        def row_body(h, c):
            base = 448 * h + 225
            k0 = 7 * h
            for j in range(_W // 16):
                r_in = (k0 + j) // 8
                c_in = 16 * ((k0 + j) % 8)
                v = ibuf[r_in, pl.ds(c_in, 16)]
                idx = (base + 32 * j) + lane2
                plsc.store_scatter(obuf, [idx], v)
            return c
        lax.fori_loop(0, _H, row_body, 0)

    def outer(tt, carry):
        for b in range(2):
            t = 2 * tt + b
            img = base_img + t
            pltpu.make_async_copy(x_hbm.at[img], inbufs[b], sem_ins[b]).wait()
            @pl.when(tt > 0)
            def _():
                pltpu.make_async_copy(
                    obufs[b], out_hbm.at[img], sem_outs[b]).wait()
            scatter_image(inbufs[b], obufs[b])
            pltpu.async_copy(obufs[b], out_hbm.at[img], sem_outs[b])
            @pl.when(t + 2 < _N_IMG)
            def _():
                pltpu.async_copy(
                    x_hbm.at[img + 2], inbufs[b], sem_ins[b])
        return carry

    lax.fori_loop(0, _N_IMG // 2, outer, 0)

    last = base_img + _N_IMG - 2
    pltpu.make_async_copy(obuf0, out_hbm.at[last], sem_out0).wait()
    pltpu.make_async_copy(obuf1, out_hbm.at[last + 1], sem_out1).wait()


def kernel(x):
    B, I, C, H, W = x.shape
    n = B * I * C
    xf = x.reshape(n, _IMG_IN // 128, 128)
    mesh = plsc.VectorSubcoreMesh(core_axis_name="c", subcore_axis_name="s")
    out = pl.kernel(
        _sc_body,
        out_type=jax.ShapeDtypeStruct((n, _IMG_OUT), jnp.float32),
        mesh=mesh,
        scratch_types=[
            pltpu.VMEM((_IMG_IN // 128, 128), jnp.float32),
            pltpu.VMEM((_IMG_IN // 128, 128), jnp.float32),
            pltpu.VMEM((_IMG_OUT,), jnp.float32),
            pltpu.VMEM((_IMG_OUT,), jnp.float32),
            pltpu.SemaphoreType.DMA,
            pltpu.SemaphoreType.DMA,
            pltpu.SemaphoreType.DMA,
            pltpu.SemaphoreType.DMA,
        ],
        compiler_params=pltpu.CompilerParams(needs_layout_passes=False),
    )(xf)
    return out.reshape(B, I, C, 2 * _H, 2 * _W)


# speedup vs baseline: 3.6931x; 3.1483x over previous
import functools

import jax
import jax.numpy as jnp
from jax import lax
from jax.experimental import pallas as pl
from jax.experimental.pallas import tpu as pltpu
from jax.experimental.pallas import tpu_sc as plsc

_H = 112
_W = 112
_NW = 32
_N_IMG = 1536 // _NW


def _sc_body(x_hbm, out_hbm, inbuf0, inbuf1, obuf0, obuf1,
             sem_in0, sem_in1, sem_out0, sem_out1):
    inbufs = (inbuf0, inbuf1)
    obufs = (obuf0, obuf1)
    sem_ins = (sem_in0, sem_in1)
    sem_outs = (sem_out0, sem_out1)

    wid = lax.axis_index("s") * 2 + lax.axis_index("c")
    base_img = wid * _N_IMG
    lane2 = 2 * lax.broadcasted_iota(jnp.int32, (16,), 0)
    zeros16 = jnp.zeros((16,), jnp.float32)

    def zero_body(i, carry):
        def zcol(j, c2):
            obuf0[i, pl.ds(j * 16, 16)] = zeros16
            obuf1[i, pl.ds(j * 16, 16)] = zeros16
            return c2
        return lax.fori_loop(0, 2 * _W // 16, zcol, carry)

    lax.fori_loop(0, _H, zero_body, 0)

    pltpu.async_copy(x_hbm.at[base_img], inbuf0, sem_in0)
    pltpu.async_copy(x_hbm.at[base_img + 1], inbuf1, sem_in1)

    def scatter_half(ibuf, obuf, c2):
        def row_body(hl, c):
            ohvec = jnp.full((16,), 2 * hl + 1, jnp.int32)
            for j in range(_W // 16):
                v = ibuf[56 * c2 + hl, pl.ds(j * 16, 16)]
                owvec = (32 * j + 1) + lane2
                plsc.store_scatter(obuf, [ohvec, owvec], v)
            return c
        lax.fori_loop(0, _H // 2, row_body, 0)

    def outer(t, carry):
        tb_sel = t % 2
        img = base_img + t
        for tb in range(2):
            @pl.when(tb_sel == tb)
            def _():
                pltpu.make_async_copy(
                    x_hbm.at[img], inbufs[tb], sem_ins[tb]).wait()
                for c2 in range(2):
                    @pl.when(t > 0)
                    def _():
                        pltpu.make_async_copy(
                            obufs[c2],
                            out_hbm.at[img, pl.ds(_H * c2, _H)],
                            sem_outs[c2]).wait()
                    scatter_half(inbufs[tb], obufs[c2], c2)
                    pltpu.async_copy(
                        obufs[c2],
                        out_hbm.at[img, pl.ds(_H * c2, _H)],
                        sem_outs[c2])
                @pl.when(t + 2 < _N_IMG)
                def _():
                    pltpu.async_copy(
                        x_hbm.at[img + 2], inbufs[tb], sem_ins[tb])
        return carry

    lax.fori_loop(0, _N_IMG, outer, 0)

    last = base_img + _N_IMG - 1
    pltpu.make_async_copy(
        obuf0, out_hbm.at[last, pl.ds(0, _H)], sem_out0).wait()
    pltpu.make_async_copy(
        obuf1, out_hbm.at[last, pl.ds(_H, _H)], sem_out1).wait()


def kernel(x):
    B, I, C, H, W = x.shape
    n = B * I * C
    xf = x.reshape(n, H, W)
    mesh = plsc.VectorSubcoreMesh(core_axis_name="c", subcore_axis_name="s")
    out = pl.kernel(
        _sc_body,
        out_type=jax.ShapeDtypeStruct((n, 2 * H, 2 * W), jnp.float32),
        mesh=mesh,
        scratch_types=[
            pltpu.VMEM((H, W), jnp.float32),
            pltpu.VMEM((H, W), jnp.float32),
            pltpu.VMEM((H, 2 * W), jnp.float32),
            pltpu.VMEM((H, 2 * W), jnp.float32),
            pltpu.SemaphoreType.DMA,
            pltpu.SemaphoreType.DMA,
            pltpu.SemaphoreType.DMA,
            pltpu.SemaphoreType.DMA,
        ],
        compiler_params=pltpu.CompilerParams(
            needs_layout_passes=False,
            use_tc_tiling_on_sc=True,
        ),
    )(xf)
    return out.reshape(B, I, C, 2 * H, 2 * W)
